# Initial kernel scaffold; baseline (speedup 1.0000x reference)
import jax
import jax.numpy as jnp
from jax.experimental import pallas as pl


def kernel(marker_values, rel_positions, cell_types, W, b, cell_type_table, position_table):
    raise NotImplementedError("write your pallas kernel here")


# speedup vs baseline: 3.5560x; 3.5560x over previous
import functools

import jax
import jax.numpy as jnp
from jax import lax
from jax.experimental import pallas as pl
from jax.experimental.pallas import tpu as pltpu
from jax.experimental.pallas import tpu_sc as plsc

BATCH = 16384
SEQ = 50
N = BATCH * SEQ
HID = 64
MRK = 16
VOC = 1000

NC = 2
NS = 16
NW = NC * NS
TPW = N // NW
TB = 128
NBLK = TPW // TB
GRP = 16
OBW = HID + 1
GW = 128


def _sc_gather_sum(rp_hbm, ct_hbm, pt_hbm, ctab_hbm, out_hbm,
                   pt_v, ctab_v, rp_v, ctv_v, ob0, ob1, sem0, sem1):
    wid = lax.axis_index("s") * NC + lax.axis_index("c")
    tok_base = wid * TPW

    pltpu.sync_copy(pt_hbm, pt_v)
    pltpu.sync_copy(ctab_hbm, ctab_v)

    iota = lax.iota(jnp.int32, GRP)
    vmax = jnp.full((GRP,), VOC - 1, jnp.int32)
    zero = jnp.zeros((GRP,), jnp.int32)
    m31 = jnp.full((GRP,), 31, jnp.int32)

    def fill_group(rx, ry, rc, ob):
        rxs, rys, rcs = rx & m31, ry & m31, rc & m31
        @plsc.parallel_loop(0, HID, unroll=8)
        def _(c):
            cc = jnp.full((GRP,), 0, jnp.int32) + c
            gx = plsc.load_gather(pt_v, [rx, cc ^ rxs])
            gy = plsc.load_gather(pt_v, [ry, cc ^ rys])
            gz = plsc.load_gather(ctab_v, [rc, cc ^ rcs])
            plsc.store_scatter(ob, [iota, cc], gx + gy + gz)

    def group_indices(t_loc):
        ridx = iota * 2 + (2 * t_loc)
        rx = plsc.load_gather(rp_v, [ridx])
        ry = plsc.load_gather(rp_v, [ridx + 1])
        rc = ctv_v[pl.ds(t_loc, GRP)]
        clip = lambda v: jnp.minimum(jnp.maximum(v, zero), vmax)
        return clip(rx), clip(ry), clip(rc)

    NPAIR = TB // (2 * GRP)

    def block_body(blk, carry):
        tok0 = tok_base + blk * TB
        pltpu.sync_copy(rp_hbm.at[pl.ds(tok0 * 2, TB * 2)], rp_v)
        pltpu.sync_copy(ct_hbm.at[pl.ds(tok0, TB)], ctv_v)

        def pair_body(pair, carry2):
            for half, (ob, sem) in enumerate(((ob0, sem0), (ob1, sem1))):
                t_loc = pair * 2 * GRP + half * GRP
                src = ob.at[:, pl.ds(0, HID)]
                dst = out_hbm.at[pl.ds(tok0 + t_loc, GRP), pl.ds(0, HID)]

                @pl.when((blk > 0) | (pair > 0))
                def _():
                    pltpu.make_async_copy(src, dst, sem).wait()

                rx, ry, rc = group_indices(t_loc)
                fill_group(rx, ry, rc, ob)
                pltpu.async_copy(src, dst, sem)
            return carry2

        return lax.fori_loop(0, NPAIR, pair_body, carry)

    lax.fori_loop(0, NBLK, block_body, 0)
    tail = out_hbm.at[pl.ds(0, GRP), pl.ds(0, HID)]
    pltpu.make_async_copy(ob0.at[:, pl.ds(0, HID)], tail, sem0).wait()
    pltpu.make_async_copy(ob1.at[:, pl.ds(0, HID)], tail, sem1).wait()


@jax.jit
def _gather_sum(rel_flat, ct_flat, pos_table_sw, cell_table_sw):
    mesh = plsc.VectorSubcoreMesh(
        core_axis_name="c", subcore_axis_name="s",
        num_cores=NC, num_subcores=NS)
    return pl.kernel(
        _sc_gather_sum,
        out_type=jax.ShapeDtypeStruct((N, GW), jnp.float32),
        mesh=mesh,
        compiler_params=pltpu.CompilerParams(
            needs_layout_passes=False, use_tc_tiling_on_sc=False),
        scratch_types=[
            pltpu.VMEM((VOC, HID), jnp.float32),
            pltpu.VMEM((VOC, HID), jnp.float32),
            pltpu.VMEM((TB * 2,), jnp.int32),
            pltpu.VMEM((TB,), jnp.int32),
            pltpu.VMEM((GRP, OBW), jnp.float32),
            pltpu.VMEM((GRP, OBW), jnp.float32),
            pltpu.SemaphoreType.DMA,
            pltpu.SemaphoreType.DMA,
        ],
    )(rel_flat, ct_flat, pos_table_sw, cell_table_sw)


def _tc_body(g_ref, m_ref, w_ref, b_ref, o_ref):
    proj = jnp.dot(m_ref[...], w_ref[...], preferred_element_type=jnp.float32)
    o_ref[...] = g_ref[:, :HID] + proj + b_ref[...]


BN = 4096


@jax.jit
def _project_add(g, markers, W, b2):
    return pl.pallas_call(
        _tc_body,
        grid=(N // BN,),
        in_specs=[
            pl.BlockSpec((BN, GW), lambda i: (i, 0)),
            pl.BlockSpec((BN, MRK), lambda i: (i, 0)),
            pl.BlockSpec((MRK, HID), lambda i: (0, 0)),
            pl.BlockSpec((1, HID), lambda i: (0, 0)),
        ],
        out_specs=pl.BlockSpec((BN, HID), lambda i: (i, 0)),
        out_shape=jax.ShapeDtypeStruct((N, HID), jnp.float32),
    )(g, markers, W, b2)


def _swizzle(table):
    r = jnp.arange(VOC, dtype=jnp.int32)[:, None]
    c = jnp.arange(HID, dtype=jnp.int32)[None, :]
    return jnp.take_along_axis(table, c ^ (r & 31), axis=1)


def kernel(marker_values, rel_positions, cell_types, W, b, cell_type_table, position_table):
    rel_flat = rel_positions.reshape(N * 2)
    ct_flat = cell_types.reshape(N)
    g = _gather_sum(rel_flat, ct_flat,
                    _swizzle(position_table), _swizzle(cell_type_table))
    out = _project_add(g, marker_values.reshape(N, MRK), W, b.reshape(1, HID))
    return out.reshape(BATCH, SEQ, HID)


# speedup vs baseline: 3.6621x; 1.0298x over previous
import functools

import jax
import jax.numpy as jnp
from jax import lax
from jax.experimental import pallas as pl
from jax.experimental.pallas import tpu as pltpu
from jax.experimental.pallas import tpu_sc as plsc

BATCH = 16384
SEQ = 50
N = BATCH * SEQ
HID = 64
MRK = 16
VOC = 1000

NC = 2
NS = 16
NW = NC * NS
TPW = N // NW
TB = 128
NBLK = TPW // TB
GRP = 16
OBW = HID + 1
GW = 128


def _sc_gather_sum(rp_hbm, ct_hbm, pt_hbm, ctab_hbm, out_hbm,
                   pt_v, ctab_v, rp_v, ctv_v, ob0, ob1, sem0, sem1):
    wid = lax.axis_index("s") * NC + lax.axis_index("c")
    tok_base = wid * TPW

    pltpu.sync_copy(pt_hbm, pt_v)
    pltpu.sync_copy(ctab_hbm, ctab_v)

    iota = lax.iota(jnp.int32, GRP)
    vmax = jnp.full((GRP,), VOC - 1, jnp.int32)
    zero = jnp.zeros((GRP,), jnp.int32)
    m31 = jnp.full((GRP,), 31, jnp.int32)

    def fill_group(rx, ry, rc, ob):
        rxs, rys, rcs = rx & m31, ry & m31, rc & m31
        @plsc.parallel_loop(0, HID, unroll=8)
        def _(c):
            cc = jnp.full((GRP,), 0, jnp.int32) + c
            gx = plsc.load_gather(pt_v, [rx, cc ^ rxs])
            gy = plsc.load_gather(pt_v, [ry, cc ^ rys])
            gz = plsc.load_gather(ctab_v, [rc, cc ^ rcs])
            plsc.store_scatter(ob, [iota, cc], gx + gy + gz)

    def group_indices(t_loc):
        ridx = iota * 2 + (2 * t_loc)
        rx = plsc.load_gather(rp_v, [ridx])
        ry = plsc.load_gather(rp_v, [ridx + 1])
        rc = ctv_v[pl.ds(t_loc, GRP)]
        clip = lambda v: jnp.minimum(jnp.maximum(v, zero), vmax)
        return clip(rx), clip(ry), clip(rc)

    NPAIR = TB // (2 * GRP)

    def block_body(blk, carry):
        tok0 = tok_base + blk * TB
        pltpu.sync_copy(rp_hbm.at[pl.ds(tok0 * 2, TB * 2)], rp_v)
        pltpu.sync_copy(ct_hbm.at[pl.ds(tok0, TB)], ctv_v)

        def pair_body(pair, carry2):
            for half, (ob, sem) in enumerate(((ob0, sem0), (ob1, sem1))):
                t_loc = pair * 2 * GRP + half * GRP
                src = ob.at[:, pl.ds(0, HID)]
                dst = out_hbm.at[pl.ds(tok0 + t_loc, GRP), pl.ds(0, HID)]

                @pl.when((blk > 0) | (pair > 0))
                def _():
                    pltpu.make_async_copy(src, dst, sem).wait()

                rx, ry, rc = group_indices(t_loc)
                fill_group(rx, ry, rc, ob)
                pltpu.async_copy(src, dst, sem)
            return carry2

        return lax.fori_loop(0, NPAIR, pair_body, carry)

    lax.fori_loop(0, NBLK, block_body, 0)
    tail = out_hbm.at[pl.ds(0, GRP), pl.ds(0, HID)]
    pltpu.make_async_copy(ob0.at[:, pl.ds(0, HID)], tail, sem0).wait()
    pltpu.make_async_copy(ob1.at[:, pl.ds(0, HID)], tail, sem1).wait()


@jax.jit
def _gather_sum(rel_flat, ct_flat, pos_table_sw, cell_table_sw):
    mesh = plsc.VectorSubcoreMesh(
        core_axis_name="c", subcore_axis_name="s",
        num_cores=NC, num_subcores=NS)
    return pl.kernel(
        _sc_gather_sum,
        out_type=jax.ShapeDtypeStruct((N, GW), jnp.float32),
        mesh=mesh,
        compiler_params=pltpu.CompilerParams(
            needs_layout_passes=False, use_tc_tiling_on_sc=False),
        scratch_types=[
            pltpu.VMEM((VOC, HID), jnp.float32),
            pltpu.VMEM((VOC, HID), jnp.float32),
            pltpu.VMEM((TB * 2,), jnp.int32),
            pltpu.VMEM((TB,), jnp.int32),
            pltpu.VMEM((GRP, OBW), jnp.float32),
            pltpu.VMEM((GRP, OBW), jnp.float32),
            pltpu.SemaphoreType.DMA,
            pltpu.SemaphoreType.DMA,
        ],
    )(rel_flat, ct_flat, pos_table_sw, cell_table_sw)


def _tc_body(g_ref, m_ref, w_ref, b_ref, o_ref):
    proj = lax.dot_general(m_ref[...], w_ref[...], (((0,), (0,)), ((), ())),
                           preferred_element_type=jnp.float32)
    o_ref[...] = g_ref[:, :HID] + proj + b_ref[...]


BN = 4096


@jax.jit
def _project_add(g, markers, W, b2):
    return pl.pallas_call(
        _tc_body,
        grid=(N // BN,),
        in_specs=[
            pl.BlockSpec((BN, GW), lambda i: (i, 0)),
            pl.BlockSpec((MRK, BN), lambda i: (0, i)),
            pl.BlockSpec((MRK, HID), lambda i: (0, 0)),
            pl.BlockSpec((1, HID), lambda i: (0, 0)),
        ],
        out_specs=pl.BlockSpec((BN, HID), lambda i: (i, 0)),
        out_shape=jax.ShapeDtypeStruct((N, HID), jnp.float32),
    )(g, markers, W, b2)


def _swizzle(table):
    r = jnp.arange(VOC, dtype=jnp.int32)[:, None]
    c = jnp.arange(HID, dtype=jnp.int32)[None, :]
    return jnp.take_along_axis(table, c ^ (r & 31), axis=1)


def kernel(marker_values, rel_positions, cell_types, W, b, cell_type_table, position_table):
    rel_flat = rel_positions.reshape(N * 2)
    ct_flat = cell_types.reshape(N)
    g = _gather_sum(rel_flat, ct_flat,
                    _swizzle(position_table), _swizzle(cell_type_table))
    out = _project_add(g, marker_values.reshape(N, MRK).T, W, b.reshape(1, HID))
    return out.reshape(BATCH, SEQ, HID)


# speedup vs baseline: 4.1194x; 1.1249x over previous
import functools

import jax
import jax.numpy as jnp
from jax import lax
from jax.experimental import pallas as pl
from jax.experimental.pallas import tpu as pltpu
from jax.experimental.pallas import tpu_sc as plsc

BATCH = 16384
SEQ = 50
N = BATCH * SEQ
HID = 64
MRK = 16
VOC = 1000

NC = 2
NS = 16
NW = NC * NS
TPW = N // NW
TB = 128
NBLK = TPW // TB
GRP = 16
OBW = HID + 1
GW = 128


def _sc_gather_sum(rp_hbm, ct_hbm, pt_hbm, ctab_hbm, out_hbm,
                   pt_v, ctab_v, rp_v, ctv_v, ob0, ob1, sem0, sem1):
    wid = lax.axis_index("s") * NC + lax.axis_index("c")
    tok_base = wid * TPW

    pltpu.sync_copy(pt_hbm, pt_v)
    pltpu.sync_copy(ctab_hbm, ctab_v)

    iota = lax.iota(jnp.int32, GRP)
    vmax = jnp.full((GRP,), VOC - 1, jnp.int32)
    zero = jnp.zeros((GRP,), jnp.int32)
    m31 = jnp.full((GRP,), 31, jnp.int32)

    def fill_group(rx, ry, rc, ob):
        rxs, rys, rcs = rx & m31, ry & m31, rc & m31
        @plsc.parallel_loop(0, HID, unroll=8)
        def _(c):
            cc = jnp.full((GRP,), 0, jnp.int32) + c
            gx = plsc.load_gather(pt_v, [rx, cc ^ rxs])
            gy = plsc.load_gather(pt_v, [ry, cc ^ rys])
            gz = plsc.load_gather(ctab_v, [rc, cc ^ rcs])
            plsc.store_scatter(ob, [iota, cc], gx + gy + gz)

    def group_indices(t_loc):
        ridx = iota * 2 + (2 * t_loc)
        rx = plsc.load_gather(rp_v, [ridx])
        ry = plsc.load_gather(rp_v, [ridx + 1])
        rc = ctv_v[pl.ds(t_loc, GRP)]
        clip = lambda v: jnp.minimum(jnp.maximum(v, zero), vmax)
        return clip(rx), clip(ry), clip(rc)

    NPAIR = TB // (2 * GRP)

    def block_body(blk, carry):
        tok0 = tok_base + blk * TB
        pltpu.sync_copy(rp_hbm.at[pl.ds(tok0 * 2, TB * 2)], rp_v)
        pltpu.sync_copy(ct_hbm.at[pl.ds(tok0, TB)], ctv_v)

        def pair_body(pair, carry2):
            for half, (ob, sem) in enumerate(((ob0, sem0), (ob1, sem1))):
                t_loc = pair * 2 * GRP + half * GRP
                src = ob.at[:, pl.ds(0, HID)]
                dst = out_hbm.at[pl.ds(tok0 + t_loc, GRP), pl.ds(0, HID)]

                @pl.when((blk > 0) | (pair > 0))
                def _():
                    pltpu.make_async_copy(src, dst, sem).wait()

                rx, ry, rc = group_indices(t_loc)
                fill_group(rx, ry, rc, ob)
                pltpu.async_copy(src, dst, sem)
            return carry2

        return lax.fori_loop(0, NPAIR, pair_body, carry)

    lax.fori_loop(0, NBLK, block_body, 0)
    tail = out_hbm.at[pl.ds(0, GRP), pl.ds(0, HID)]
    pltpu.make_async_copy(ob0.at[:, pl.ds(0, HID)], tail, sem0).wait()
    pltpu.make_async_copy(ob1.at[:, pl.ds(0, HID)], tail, sem1).wait()


@jax.jit
def _gather_sum(rel_flat, ct_flat, pos_table_sw, cell_table_sw):
    mesh = plsc.VectorSubcoreMesh(
        core_axis_name="c", subcore_axis_name="s",
        num_cores=NC, num_subcores=NS)
    return pl.kernel(
        _sc_gather_sum,
        out_type=jax.ShapeDtypeStruct((N, GW), jnp.float32),
        mesh=mesh,
        compiler_params=pltpu.CompilerParams(
            needs_layout_passes=False, use_tc_tiling_on_sc=False),
        scratch_types=[
            pltpu.VMEM((VOC, HID), jnp.float32),
            pltpu.VMEM((VOC, HID), jnp.float32),
            pltpu.VMEM((TB * 2,), jnp.int32),
            pltpu.VMEM((TB,), jnp.int32),
            pltpu.VMEM((GRP, OBW), jnp.float32),
            pltpu.VMEM((GRP, OBW), jnp.float32),
            pltpu.SemaphoreType.DMA,
            pltpu.SemaphoreType.DMA,
        ],
    )(rel_flat, ct_flat, pos_table_sw, cell_table_sw)


def _tc_body(g_ref, m_ref, w_ref, b_ref, o_ref):
    proj = lax.dot_general(m_ref[...], w_ref[...], (((0,), (0,)), ((), ())),
                           preferred_element_type=jnp.float32)
    res = g_ref[:, :HID] + proj + b_ref[...]
    o_ref[...] = res.reshape(BB, SEQ, HID)


BB = 128
BN = BB * SEQ


@jax.jit
def _project_add(g, markers, W, b2):
    return pl.pallas_call(
        _tc_body,
        grid=(BATCH // BB,),
        in_specs=[
            pl.BlockSpec((BN, GW), lambda i: (i, 0)),
            pl.BlockSpec((MRK, BN), lambda i: (0, i)),
            pl.BlockSpec((MRK, HID), lambda i: (0, 0)),
            pl.BlockSpec((1, HID), lambda i: (0, 0)),
        ],
        out_specs=pl.BlockSpec((BB, SEQ, HID), lambda i: (i, 0, 0)),
        out_shape=jax.ShapeDtypeStruct((BATCH, SEQ, HID), jnp.float32),
    )(g, markers, W, b2)


def _swizzle(table):
    r = jnp.arange(VOC, dtype=jnp.int32)[:, None]
    c = jnp.arange(HID, dtype=jnp.int32)[None, :]
    return jnp.take_along_axis(table, c ^ (r & 31), axis=1)


def kernel(marker_values, rel_positions, cell_types, W, b, cell_type_table, position_table):
    rel_flat = rel_positions.reshape(N * 2)
    ct_flat = cell_types.reshape(N)
    g = _gather_sum(rel_flat, ct_flat,
                    _swizzle(position_table), _swizzle(cell_type_table))
    return _project_add(g, marker_values.reshape(N, MRK).T, W,
                        b.reshape(1, HID))


# speedup vs baseline: 4.3669x; 1.0601x over previous
import functools

import jax
import jax.numpy as jnp
from jax import lax
from jax.experimental import pallas as pl
from jax.experimental.pallas import tpu as pltpu
from jax.experimental.pallas import tpu_sc as plsc

BATCH = 16384
SEQ = 50
N = BATCH * SEQ
HID = 64
MRK = 16
VOC = 1000

NC = 2
NS = 16
NW = NC * NS
NCH = 2
NT = N // NCH
BCH = BATCH // NCH
TPW = NT // NW
TB = 128
NBLK = TPW // TB
GRP = 16
OBW = HID + 1
GW = 128


def _sc_gather_sum(rp_hbm, ct_hbm, pt_hbm, ctab_hbm, out_hbm,
                   pt_v, ctab_v, rp_v, ctv_v, ob0, ob1, sem0, sem1):
    wid = lax.axis_index("s") * NC + lax.axis_index("c")
    tok_base = wid * TPW

    pltpu.sync_copy(pt_hbm, pt_v)
    pltpu.sync_copy(ctab_hbm, ctab_v)

    iota = lax.iota(jnp.int32, GRP)
    vmax = jnp.full((GRP,), VOC - 1, jnp.int32)
    zero = jnp.zeros((GRP,), jnp.int32)
    m31 = jnp.full((GRP,), 31, jnp.int32)

    def fill_group(rx, ry, rc, ob):
        rxs, rys, rcs = rx & m31, ry & m31, rc & m31
        @plsc.parallel_loop(0, HID, unroll=8)
        def _(c):
            cc = jnp.full((GRP,), 0, jnp.int32) + c
            gx = plsc.load_gather(pt_v, [rx, cc ^ rxs])
            gy = plsc.load_gather(pt_v, [ry, cc ^ rys])
            gz = plsc.load_gather(ctab_v, [rc, cc ^ rcs])
            plsc.store_scatter(ob, [iota, cc], gx + gy + gz)

    def group_indices(t_loc):
        ridx = iota * 2 + (2 * t_loc)
        rx = plsc.load_gather(rp_v, [ridx])
        ry = plsc.load_gather(rp_v, [ridx + 1])
        rc = ctv_v[pl.ds(t_loc, GRP)]
        clip = lambda v: jnp.minimum(jnp.maximum(v, zero), vmax)
        return clip(rx), clip(ry), clip(rc)

    NPAIR = TB // (2 * GRP)

    def block_body(blk, carry):
        tok0 = tok_base + blk * TB
        pltpu.sync_copy(rp_hbm.at[pl.ds(tok0 * 2, TB * 2)], rp_v)
        pltpu.sync_copy(ct_hbm.at[pl.ds(tok0, TB)], ctv_v)

        def pair_body(pair, carry2):
            for half, (ob, sem) in enumerate(((ob0, sem0), (ob1, sem1))):
                t_loc = pair * 2 * GRP + half * GRP
                src = ob.at[:, pl.ds(0, HID)]
                dst = out_hbm.at[pl.ds(tok0 + t_loc, GRP), pl.ds(0, HID)]

                @pl.when((blk > 0) | (pair > 0))
                def _():
                    pltpu.make_async_copy(src, dst, sem).wait()

                rx, ry, rc = group_indices(t_loc)
                fill_group(rx, ry, rc, ob)
                pltpu.async_copy(src, dst, sem)
            return carry2

        return lax.fori_loop(0, NPAIR, pair_body, carry)

    lax.fori_loop(0, NBLK, block_body, 0)
    tail = out_hbm.at[pl.ds(0, GRP), pl.ds(0, HID)]
    pltpu.make_async_copy(ob0.at[:, pl.ds(0, HID)], tail, sem0).wait()
    pltpu.make_async_copy(ob1.at[:, pl.ds(0, HID)], tail, sem1).wait()


@jax.jit
def _gather_sum(rel_flat, ct_flat, pos_table_sw, cell_table_sw):
    mesh = plsc.VectorSubcoreMesh(
        core_axis_name="c", subcore_axis_name="s",
        num_cores=NC, num_subcores=NS)
    return pl.kernel(
        _sc_gather_sum,
        out_type=jax.ShapeDtypeStruct((NT, GW), jnp.float32),
        mesh=mesh,
        compiler_params=pltpu.CompilerParams(
            needs_layout_passes=False, use_tc_tiling_on_sc=False),
        scratch_types=[
            pltpu.VMEM((VOC, HID), jnp.float32),
            pltpu.VMEM((VOC, HID), jnp.float32),
            pltpu.VMEM((TB * 2,), jnp.int32),
            pltpu.VMEM((TB,), jnp.int32),
            pltpu.VMEM((GRP, OBW), jnp.float32),
            pltpu.VMEM((GRP, OBW), jnp.float32),
            pltpu.SemaphoreType.DMA,
            pltpu.SemaphoreType.DMA,
        ],
    )(rel_flat, ct_flat, pos_table_sw, cell_table_sw)


def _tc_body(g_ref, m_ref, w_ref, b_ref, o_ref):
    proj = lax.dot_general(m_ref[...], w_ref[...], (((0,), (0,)), ((), ())),
                           preferred_element_type=jnp.float32)
    res = g_ref[:, :HID] + proj + b_ref[...]
    o_ref[...] = res.reshape(BB, SEQ, HID)


BB = 128
BN = BB * SEQ


def _tc_body0(g_ref, m_ref, w_ref, b_ref, o_ref):
    _tc_body(g_ref, m_ref, w_ref, b_ref, o_ref)


def _tc_body1(o_in_ref, g_ref, m_ref, w_ref, b_ref, o_ref):
    _tc_body(g_ref, m_ref, w_ref, b_ref, o_ref)


def _project_add(chunk, out_prev, g, markers, W, b2):
    base = chunk * (BCH // BB)
    out_sds = jax.ShapeDtypeStruct((BATCH, SEQ, HID), jnp.float32)
    o_spec = pl.BlockSpec((BB, SEQ, HID), lambda i: (i + base, 0, 0))
    specs = [
        pl.BlockSpec((BN, GW), lambda i: (i, 0)),
        pl.BlockSpec((MRK, BN), lambda i: (0, i)),
        pl.BlockSpec((MRK, HID), lambda i: (0, 0)),
        pl.BlockSpec((1, HID), lambda i: (0, 0)),
    ]
    if chunk == 0:
        return pl.pallas_call(
            _tc_body0, grid=(BCH // BB,), in_specs=specs,
            out_specs=o_spec, out_shape=out_sds,
        )(g, markers, W, b2)
    return pl.pallas_call(
        _tc_body1, grid=(BCH // BB,),
        in_specs=[pl.BlockSpec((8, SEQ, HID), lambda i: (0, 0, 0))] + specs,
        out_specs=o_spec, out_shape=out_sds,
        input_output_aliases={0: 0},
    )(out_prev, g, markers, W, b2)


def _swizzle(table):
    r = jnp.arange(VOC, dtype=jnp.int32)[:, None]
    c = jnp.arange(HID, dtype=jnp.int32)[None, :]
    return jnp.take_along_axis(table, c ^ (r & 31), axis=1)


def kernel(marker_values, rel_positions, cell_types, W, b, cell_type_table, position_table):
    rel_flat = rel_positions.reshape(N * 2)
    ct_flat = cell_types.reshape(N)
    mt = marker_values.reshape(N, MRK).T
    pt_sw, ct_sw = _swizzle(position_table), _swizzle(cell_type_table)
    b2 = b.reshape(1, HID)
    gs = [_gather_sum(lax.slice(rel_flat, (c * NT * 2,), ((c + 1) * NT * 2,)),
                      lax.slice(ct_flat, (c * NT,), ((c + 1) * NT,)),
                      pt_sw, ct_sw)
          for c in range(NCH)]
    out = None
    for c in range(NCH):
        mt_c = lax.slice(mt, (0, c * NT), (MRK, (c + 1) * NT))
        out = _project_add(c, out, gs[c], mt_c, W, b2)
    return out


# speedup vs baseline: 4.3900x; 1.0053x over previous
import functools

import jax
import jax.numpy as jnp
from jax import lax
from jax.experimental import pallas as pl
from jax.experimental.pallas import tpu as pltpu
from jax.experimental.pallas import tpu_sc as plsc

BATCH = 16384
SEQ = 50
N = BATCH * SEQ
HID = 64
MRK = 16
VOC = 1000

NC = 2
NS = 16
NW = NC * NS
NCH = 4
NT = N // NCH
BCH = BATCH // NCH
TPW = NT // NW
TB = 128
NBLK = TPW // TB
GRP = 16
OBW = HID + 1
GW = 128


def _sc_gather_sum(rp_hbm, ct_hbm, pt_hbm, ctab_hbm, out_hbm,
                   pt_v, ctab_v, rp_v, ctv_v, ob0, ob1, sem0, sem1):
    wid = lax.axis_index("s") * NC + lax.axis_index("c")
    tok_base = wid * TPW

    pltpu.sync_copy(pt_hbm, pt_v)
    pltpu.sync_copy(ctab_hbm, ctab_v)

    iota = lax.iota(jnp.int32, GRP)
    vmax = jnp.full((GRP,), VOC - 1, jnp.int32)
    zero = jnp.zeros((GRP,), jnp.int32)
    m31 = jnp.full((GRP,), 31, jnp.int32)

    def fill_group(rx, ry, rc, ob):
        rxs, rys, rcs = rx & m31, ry & m31, rc & m31
        @plsc.parallel_loop(0, HID, unroll=8)
        def _(c):
            cc = jnp.full((GRP,), 0, jnp.int32) + c
            gx = plsc.load_gather(pt_v, [rx, cc ^ rxs])
            gy = plsc.load_gather(pt_v, [ry, cc ^ rys])
            gz = plsc.load_gather(ctab_v, [rc, cc ^ rcs])
            plsc.store_scatter(ob, [iota, cc], gx + gy + gz)

    def group_indices(t_loc):
        ridx = iota * 2 + (2 * t_loc)
        rx = plsc.load_gather(rp_v, [ridx])
        ry = plsc.load_gather(rp_v, [ridx + 1])
        rc = ctv_v[pl.ds(t_loc, GRP)]
        clip = lambda v: jnp.minimum(jnp.maximum(v, zero), vmax)
        return clip(rx), clip(ry), clip(rc)

    NPAIR = TB // (2 * GRP)

    def block_body(blk, carry):
        tok0 = tok_base + blk * TB
        pltpu.sync_copy(rp_hbm.at[pl.ds(tok0 * 2, TB * 2)], rp_v)
        pltpu.sync_copy(ct_hbm.at[pl.ds(tok0, TB)], ctv_v)

        def pair_body(pair, carry2):
            for half, (ob, sem) in enumerate(((ob0, sem0), (ob1, sem1))):
                t_loc = pair * 2 * GRP + half * GRP
                src = ob.at[:, pl.ds(0, HID)]
                dst = out_hbm.at[pl.ds(tok0 + t_loc, GRP), pl.ds(0, HID)]

                @pl.when((blk > 0) | (pair > 0))
                def _():
                    pltpu.make_async_copy(src, dst, sem).wait()

                rx, ry, rc = group_indices(t_loc)
                fill_group(rx, ry, rc, ob)
                pltpu.async_copy(src, dst, sem)
            return carry2

        return lax.fori_loop(0, NPAIR, pair_body, carry)

    lax.fori_loop(0, NBLK, block_body, 0)
    tail = out_hbm.at[pl.ds(0, GRP), pl.ds(0, HID)]
    pltpu.make_async_copy(ob0.at[:, pl.ds(0, HID)], tail, sem0).wait()
    pltpu.make_async_copy(ob1.at[:, pl.ds(0, HID)], tail, sem1).wait()


@jax.jit
def _gather_sum(rel_flat, ct_flat, pos_table_sw, cell_table_sw):
    mesh = plsc.VectorSubcoreMesh(
        core_axis_name="c", subcore_axis_name="s",
        num_cores=NC, num_subcores=NS)
    return pl.kernel(
        _sc_gather_sum,
        out_type=jax.ShapeDtypeStruct((NT, GW), jnp.float32),
        mesh=mesh,
        compiler_params=pltpu.CompilerParams(
            needs_layout_passes=False, use_tc_tiling_on_sc=False),
        scratch_types=[
            pltpu.VMEM((VOC, HID), jnp.float32),
            pltpu.VMEM((VOC, HID), jnp.float32),
            pltpu.VMEM((TB * 2,), jnp.int32),
            pltpu.VMEM((TB,), jnp.int32),
            pltpu.VMEM((GRP, OBW), jnp.float32),
            pltpu.VMEM((GRP, OBW), jnp.float32),
            pltpu.SemaphoreType.DMA,
            pltpu.SemaphoreType.DMA,
        ],
    )(rel_flat, ct_flat, pos_table_sw, cell_table_sw)


def _tc_body(g_ref, m_ref, w_ref, b_ref, o_ref):
    proj = lax.dot_general(m_ref[...], w_ref[...], (((0,), (0,)), ((), ())),
                           preferred_element_type=jnp.float32)
    res = g_ref[:, :HID] + proj + b_ref[...]
    o_ref[...] = res.reshape(BB, SEQ, HID)


BB = 128
BN = BB * SEQ


def _tc_body0(g_ref, m_ref, w_ref, b_ref, o_ref):
    _tc_body(g_ref, m_ref, w_ref, b_ref, o_ref)


def _tc_body1(o_in_ref, g_ref, m_ref, w_ref, b_ref, o_ref):
    _tc_body(g_ref, m_ref, w_ref, b_ref, o_ref)


def _project_add(chunk, out_prev, g, markers, W, b2):
    base = chunk * (BCH // BB)
    out_sds = jax.ShapeDtypeStruct((BATCH, SEQ, HID), jnp.float32)
    o_spec = pl.BlockSpec((BB, SEQ, HID), lambda i: (i + base, 0, 0))
    specs = [
        pl.BlockSpec((BN, GW), lambda i: (i, 0)),
        pl.BlockSpec((MRK, BN), lambda i: (0, i)),
        pl.BlockSpec((MRK, HID), lambda i: (0, 0)),
        pl.BlockSpec((1, HID), lambda i: (0, 0)),
    ]
    if chunk == 0:
        return pl.pallas_call(
            _tc_body0, grid=(BCH // BB,), in_specs=specs,
            out_specs=o_spec, out_shape=out_sds,
        )(g, markers, W, b2)
    return pl.pallas_call(
        _tc_body1, grid=(BCH // BB,),
        in_specs=[pl.BlockSpec((8, SEQ, HID), lambda i: (0, 0, 0))] + specs,
        out_specs=o_spec, out_shape=out_sds,
        input_output_aliases={0: 0},
    )(out_prev, g, markers, W, b2)


def _swizzle(table):
    r = jnp.arange(VOC, dtype=jnp.int32)[:, None]
    c = jnp.arange(HID, dtype=jnp.int32)[None, :]
    return jnp.take_along_axis(table, c ^ (r & 31), axis=1)


def kernel(marker_values, rel_positions, cell_types, W, b, cell_type_table, position_table):
    rel_flat = rel_positions.reshape(N * 2)
    ct_flat = cell_types.reshape(N)
    mt = marker_values.reshape(N, MRK).T
    pt_sw, ct_sw = _swizzle(position_table), _swizzle(cell_type_table)
    b2 = b.reshape(1, HID)
    gs = [_gather_sum(lax.slice(rel_flat, (c * NT * 2,), ((c + 1) * NT * 2,)),
                      lax.slice(ct_flat, (c * NT,), ((c + 1) * NT,)),
                      pt_sw, ct_sw)
          for c in range(NCH)]
    out = None
    for c in range(NCH):
        mt_c = lax.slice(mt, (0, c * NT), (MRK, (c + 1) * NT))
        out = _project_add(c, out, gs[c], mt_c, W, b2)
    return out
